# baseline (speedup 1.0000x reference)
import functools

import jax
import jax.numpy as jnp
from jax import lax
from jax.experimental import pallas as pl
from jax.experimental.pallas import tpu as pltpu
from jax.experimental.pallas import tpu_sc as plsc

_HID = 1024
_NC = 2
_NS = 16
_NW = _NC * _NS
_C = 32


def kernel(segments, table):
    batch, seq = segments.shape
    n = batch * seq
    b_per_w = n // _NW
    nch = b_per_w // _C
    idx = segments.reshape(n).astype(jnp.int32)
    mesh = plsc.VectorSubcoreMesh(core_axis_name="c", subcore_axis_name="s")

    @functools.partial(
        pl.kernel,
        mesh=mesh,
        out_type=jax.ShapeDtypeStruct((n, _HID), jnp.float32),
        scratch_types=[
            pltpu.VMEM((b_per_w,), jnp.int32),
            pltpu.VMEM((_C, _HID), jnp.float32),
            pltpu.VMEM((_C, _HID), jnp.float32),
            pltpu.SemaphoreType.DMA,
            pltpu.SemaphoreType.DMA,
            pltpu.SemaphoreType.DMA,
            pltpu.SemaphoreType.DMA,
        ],
    )
    def gather_kernel(tab_hbm, idx_hbm, out_hbm, idx_v, rows0, rows1,
                      gsem0, gsem1, wsem0, wsem1):
        wid = lax.axis_index("s") * _NC + lax.axis_index("c")
        base = wid * b_per_w
        pltpu.sync_copy(idx_hbm.at[pl.ds(base, b_per_w)], idx_v)

        def gather(j, rows, gsem):
            return pltpu.async_copy(
                tab_hbm.at[idx_v.at[pl.ds(j * _C, _C)]], rows, gsem)

        def write(j, rows, wsem):
            return pltpu.async_copy(
                rows, out_hbm.at[pl.ds(base + j * _C, _C)], wsem)

        def wait_write(rows, wsem):
            pltpu.make_async_copy(
                rows, out_hbm.at[pl.ds(base, _C)], wsem).wait()

        gather(0, rows0, gsem0).wait()
        write(0, rows0, wsem0)
        gather(1, rows1, gsem1).wait()
        write(1, rows1, wsem1)

        @pl.loop(2, nch, step=2)
        def _(j):
            wait_write(rows0, wsem0)
            gather(j, rows0, gsem0).wait()
            write(j, rows0, wsem0)
            wait_write(rows1, wsem1)
            gather(j + 1, rows1, gsem1).wait()
            write(j + 1, rows1, wsem1)

        wait_write(rows0, wsem0)
        wait_write(rows1, wsem1)

    return gather_kernel(table, idx).reshape(batch, seq, _HID)


# speedup vs baseline: 4.3021x; 4.3021x over previous
import functools

import jax
import jax.numpy as jnp
from jax import lax
from jax.experimental import pallas as pl
from jax.experimental.pallas import tpu as pltpu
from jax.experimental.pallas import tpu_sc as plsc

_HID = 1024
_NC = 2
_NS = 16
_NW = _NC * _NS
_C = 32


def kernel(segments, table):
    batch, seq = segments.shape
    n = batch * seq
    b_per_w = n // _NW
    nch = b_per_w // _C
    idx = segments.reshape(n).astype(jnp.int32)
    rep_table = jnp.broadcast_to(table[None], (_NW, 2, _HID))
    mesh = plsc.VectorSubcoreMesh(core_axis_name="c", subcore_axis_name="s")

    @functools.partial(
        pl.kernel,
        mesh=mesh,
        out_type=jax.ShapeDtypeStruct((n, _HID), jnp.float32),
        scratch_types=[
            pltpu.VMEM((b_per_w,), jnp.int32),
            pltpu.VMEM((_C, _HID), jnp.float32),
            pltpu.VMEM((_C, _HID), jnp.float32),
            pltpu.SemaphoreType.DMA,
            pltpu.SemaphoreType.DMA,
            pltpu.SemaphoreType.DMA,
            pltpu.SemaphoreType.DMA,
        ],
    )
    def gather_kernel(rep_hbm, idx_hbm, out_hbm, idx_v, rows0, rows1,
                      gsem0, gsem1, wsem0, wsem1):
        wid = lax.axis_index("s") * _NC + lax.axis_index("c")
        base = wid * b_per_w
        tab_hbm = rep_hbm.at[wid]
        pltpu.sync_copy(idx_hbm.at[pl.ds(base, b_per_w)], idx_v)

        def gather(j, rows, gsem):
            return pltpu.async_copy(
                tab_hbm.at[idx_v.at[pl.ds(j * _C, _C)]], rows, gsem)

        def write(j, rows, wsem):
            return pltpu.async_copy(
                rows, out_hbm.at[pl.ds(base + j * _C, _C)], wsem)

        def wait_write(rows, wsem):
            pltpu.make_async_copy(
                rows, out_hbm.at[pl.ds(base, _C)], wsem).wait()

        gather(0, rows0, gsem0).wait()
        write(0, rows0, wsem0)
        gather(1, rows1, gsem1).wait()
        write(1, rows1, wsem1)

        @pl.loop(2, nch, step=2)
        def _(j):
            wait_write(rows0, wsem0)
            gather(j, rows0, gsem0).wait()
            write(j, rows0, wsem0)
            wait_write(rows1, wsem1)
            gather(j + 1, rows1, gsem1).wait()
            write(j + 1, rows1, wsem1)

        wait_write(rows0, wsem0)
        wait_write(rows1, wsem1)

    return gather_kernel(rep_table, idx).reshape(batch, seq, _HID)


# speedup vs baseline: 19.1868x; 4.4598x over previous
import jax
import jax.numpy as jnp
from jax.experimental import pallas as pl
from jax.experimental.pallas import tpu as pltpu

_HID = 1024
_BLK = 4096


def _select_body(seg_ref, tab_ref, out_ref):
    seg = seg_ref[0, 0, :]
    segf = seg.astype(jnp.float32)[:, None]
    t0 = tab_ref[0, :][None, :]
    t1 = tab_ref[1, :][None, :]
    out_ref[...] = t0 + segf * (t1 - t0)


def kernel(segments, table):
    batch, seq = segments.shape
    n = batch * seq
    nblk = n // _BLK
    seg3 = segments.reshape(nblk, 1, _BLK).astype(jnp.int32)

    out = pl.pallas_call(
        _select_body,
        grid=(nblk,),
        in_specs=[
            pl.BlockSpec((1, 1, _BLK), lambda i: (i, 0, 0)),
            pl.BlockSpec((2, _HID), lambda i: (0, 0)),
        ],
        out_specs=pl.BlockSpec((_BLK, _HID), lambda i: (i, 0)),
        out_shape=jax.ShapeDtypeStruct((n, _HID), jnp.float32),
    )(seg3, table)
    return out.reshape(batch, seq, _HID)


# speedup vs baseline: 20.3484x; 1.0605x over previous
import jax
import jax.numpy as jnp
from jax.experimental import pallas as pl
from jax.experimental.pallas import tpu as pltpu

_HID = 1024
_BLK = 1024


def _select_body(seg_ref, tab_ref, out_ref):
    seg = seg_ref[0, 0, :]
    mask = seg[:, None] == 0
    t0 = tab_ref[0, :][None, :]
    t1 = tab_ref[1, :][None, :]
    out_ref[...] = jnp.where(mask, t0, t1)


def kernel(segments, table):
    batch, seq = segments.shape
    n = batch * seq
    nblk = n // _BLK
    seg3 = segments.reshape(nblk, 1, _BLK).astype(jnp.int32)

    out = pl.pallas_call(
        _select_body,
        grid=(nblk,),
        in_specs=[
            pl.BlockSpec((1, 1, _BLK), lambda i: (i, 0, 0)),
            pl.BlockSpec((2, _HID), lambda i: (0, 0)),
        ],
        out_specs=pl.BlockSpec((_BLK, _HID), lambda i: (i, 0)),
        out_shape=jax.ShapeDtypeStruct((n, _HID), jnp.float32),
    )(seg3, table)
    return out.reshape(batch, seq, _HID)
